# baseline (speedup 1.0000x reference)
import functools

import jax
import jax.numpy as jnp
from jax import lax
from jax.experimental import pallas as pl
from jax.experimental.pallas import tpu as pltpu
from jax.experimental.pallas import tpu_sc as plsc

_DELTA_V = 0.5
_DELTA_D = 3.0
_NLAB = 5

_B = 8
_D = 4
_N = 512 * 512
_W = 32
_WPB = _W // _B
_PPW = _N // _WPB
_ROWS = 16
_CH = _ROWS * 512
_NCH = _PPW // _CH
_NF = 6
_NST = _NLAB * _NF
_STRIDE = 7
_FACC = 16 * _STRIDE
_RES = _NST * 16


def _sc_body(pred_hbm, bin_hbm, inst_hbm, out_hbm,
             p0a, p1a, p2a, p3a, bna, ina,
             p0b, p1b, p2b, p3b, bnb, inb,
             a0, a1, a2, a3, res, sem0, sem1):
    cid = lax.axis_index("c")
    sid = lax.axis_index("s")
    wid = sid * 2 + cid
    b = wid // _WPB
    sl = wid % _WPB
    row0 = sl * (512 // _WPB)
    sems = [sem0, sem1]
    fbufs = [[p0a, p1a, p2a, p3a, bna], [p0b, p1b, p2b, p3b, bnb]]
    ibufs = [ina, inb]
    faccs = [a0, a1, a2, a3]

    def start(k, slot):
        rs = row0 + k * _ROWS
        cps = []
        for d in range(_D):
            cps.append(pltpu.async_copy(
                pred_hbm.at[b, d, pl.ds(rs, _ROWS), :],
                fbufs[slot][d], sems[slot]))
        cps.append(pltpu.async_copy(
            bin_hbm.at[b, pl.ds(rs, _ROWS), :], fbufs[slot][_D], sems[slot]))
        cps.append(pltpu.async_copy(
            inst_hbm.at[b, pl.ds(rs, _ROWS), :], ibufs[slot], sems[slot]))
        return cps

    zero16 = jnp.zeros((16,), jnp.float32)
    for fb in faccs:
        for i in range(_FACC // 16):
            fb[pl.ds(i * 16, 16)] = zero16
    lane_base = jnp.arange(16, dtype=jnp.int32) * _STRIDE

    zero16i = jnp.zeros((16,), jnp.int32)
    vacc = (zero16,) * _NLAB + (zero16i,) * _NLAB

    cps = start(0, 0)
    for k in range(_NCH):
        nxt = start(k + 1, (k + 1) % 2) if k + 1 < _NCH else []
        for cp in cps:
            cp.wait()
        slot = k % 2

        @plsc.parallel_loop(0, _CH // 16, unroll=8, carry=vacc)
        def gbody(g, cv, slot=slot):
            r = g >> 5
            o16 = (g & 31) * 16
            iv = ibufs[slot][r, pl.ds(o16, 16)]
            bv = fbufs[slot][_D][r, pl.ds(o16, 16)]
            xs = [fbufs[slot][d][r, pl.ds(o16, 16)] * bv for d in range(_D)]
            q = xs[0] * xs[0] + xs[1] * xs[1] + xs[2] * xs[2] + xs[3] * xs[3]
            idx = lane_base + iv
            for d in range(_D):
                plsc.addupdate_scatter(faccs[d], [idx], xs[d])
            new = list(cv)
            for cl in range(_NLAB):
                m = iv == cl
                new[cl] = new[cl] + jnp.where(m, q, 0.0)
                new[_NLAB + cl] = new[_NLAB + cl] + plsc.all_reduce_population_count(m)
            return tuple(new)

        vacc = gbody
        cps = nxt

    for cl in range(_NLAB):
        for d in range(_D):
            v = plsc.load_gather(faccs[d], [lane_base + cl])
            res[pl.ds((cl * _NF + d) * 16, 16)] = v
        res[pl.ds((cl * _NF + _D) * 16, 16)] = vacc[cl]
        cnt_f = vacc[_NLAB + cl].astype(jnp.float32)
        res[pl.ds((cl * _NF + _D + 1) * 16, 16)] = cnt_f * (1.0 / 16.0)
    pltpu.sync_copy(res, out_hbm.at[pl.ds(wid * _RES, _RES)])


def _stage1(pred, binary_label, inst_i32):
    mesh = plsc.VectorSubcoreMesh(core_axis_name="c", subcore_axis_name="s")
    fslot = ([pltpu.VMEM((_ROWS, 512), jnp.float32)] * (_D + 1)
             + [pltpu.VMEM((_ROWS, 512), jnp.int32)])
    return pl.kernel(
        _sc_body,
        out_type=jax.ShapeDtypeStruct((_W * _RES,), jnp.float32),
        mesh=mesh,
        compiler_params=pltpu.CompilerParams(
            needs_layout_passes=False, use_tc_tiling_on_sc=True),
        scratch_types=fslot + fslot
        + [pltpu.VMEM((_FACC,), jnp.float32)] * _D
        + [
            pltpu.VMEM((_RES,), jnp.float32),
            pltpu.SemaphoreType.DMA,
            pltpu.SemaphoreType.DMA,
        ],
    )(pred, binary_label, inst_i32)


def _epilogue_body(p_ref, o_ref):
    P = p_ref[...]
    Ps = jnp.sum(P, axis=2)
    rows = lax.broadcasted_iota(jnp.int32, (_B, _W), 0)
    cols = lax.broadcasted_iota(jnp.int32, (_B, _W), 1)
    M = (cols // _WPB == rows).astype(jnp.float32)
    stats = jnp.dot(M, Ps, preferred_element_type=jnp.float32)

    def col(k):
        return stats[:, k:k + 1]

    cnt = [col(cl * _NF + 5) for cl in range(_NLAB)]
    qv = [col(cl * _NF + 4) for cl in range(_NLAB)]
    sv = [[col(cl * _NF + d) for d in range(_D)] for cl in range(_NLAB)]

    present = [cnt[cl] > 0.0 for cl in range(_NLAB)]
    pf = [jnp.where(present[cl], 1.0, 0.0) for cl in range(_NLAB)]
    var_sum = jnp.zeros((_B, 1), jnp.float32)
    mu = []
    for cl in range(_NLAB):
        cs = jnp.maximum(cnt[cl], 1.0)
        m = [sv[cl][d] / cs for d in range(_D)]
        mu.append(m)
        mdots = sum(m[d] * sv[cl][d] for d in range(_D))
        msq = sum(m[d] * m[d] for d in range(_D))
        sumsq = qv[cl] - 2.0 * mdots + cnt[cl] * msq
        nrm = jnp.sqrt(jnp.maximum(sumsq, 0.0))
        delta = jnp.where(nrm > _DELTA_V, nrm - _DELTA_V, 0.0)
        var_sum = var_sum + jnp.where(present[cl], delta * delta, 0.0)

    Cf = jnp.zeros((_B, 1), jnp.float32)
    for cl in range(_NLAB):
        Cf = jnp.maximum(Cf, jnp.where(present[cl], float(cl), 0.0))
    L_var = var_sum / Cf

    running = jnp.zeros((_B, 1), jnp.float32)
    rank = []
    for cl in range(_NLAB):
        running = running + pf[cl]
        rank.append(running - 1.0)
    npres = running
    mr = []
    for r in range(_NLAB):
        md = [jnp.zeros((_B, 1), jnp.float32) for _ in range(_D)]
        for cl in range(_NLAB):
            selw = jnp.where((rank[cl] == float(r)) & present[cl], 1.0, 0.0)
            for d in range(_D):
                md[d] = md[d] + mu[cl][d] * selw
        mr.append(md)

    dist_sum = jnp.zeros((_B, 1), jnp.float32)
    for a in range(_NLAB):
        for b2 in range(a + 1, _NLAB):
            dsq = sum(jnp.square(mr[a][d] - mr[b2][d]) for d in range(_D))
            dd = jnp.sqrt(dsq)
            term = jnp.square(jnp.maximum(_DELTA_D - dd, 0.0))
            valid = (Cf > float(a)) & (Cf > float(b2)) & (npres > 1.5)
            dist_sum = dist_sum + 2.0 * jnp.where(valid, term, 0.0)

    total = jnp.sum(L_var + dist_sum) / float(_B)
    o_ref[...] = jnp.reshape(total, (1, 1))


def _stage2(partials):
    return pl.pallas_call(
        _epilogue_body,
        out_shape=jax.ShapeDtypeStruct((1, 1), jnp.float32),
    )(partials)


@jax.jit
def kernel(pred, binary_label, instance_label):
    partials = _stage1(pred, binary_label, instance_label.astype(jnp.int32))
    out = _stage2(partials.reshape(_W, _NST, 16))
    return out[0, 0]


# speedup vs baseline: 1.1853x; 1.1853x over previous
import functools

import jax
import jax.numpy as jnp
from jax import lax
from jax.experimental import pallas as pl
from jax.experimental.pallas import tpu as pltpu
from jax.experimental.pallas import tpu_sc as plsc

_DELTA_V = 0.5
_DELTA_D = 3.0
_NLAB = 5

_B = 8
_D = 4
_N = 512 * 512
_W = 32
_WPB = _W // _B
_PPW = _N // _WPB
_ROWS = 16
_CH = _ROWS * 512
_NCH = _PPW // _CH
_NF = 6
_NST = _NLAB * _NF
_STRIDE = 7
_FACC = 16 * _STRIDE
_RES = _NST * 16


def _sc_body(pred_hbm, bin_hbm, inst_hbm, out_hbm,
             p0a, p1a, p2a, p3a, bna, ina,
             p0b, p1b, p2b, p3b, bnb, inb,
             a0, a1, a2, a3, res, sem0, sem1):
    cid = lax.axis_index("c")
    sid = lax.axis_index("s")
    wid = sid * 2 + cid
    b = wid // _WPB
    sl = wid % _WPB
    row0 = sl * (512 // _WPB)
    sems = [sem0, sem1]
    fbufs = [[p0a, p1a, p2a, p3a, bna], [p0b, p1b, p2b, p3b, bnb]]
    ibufs = [ina, inb]
    faccs = [a0, a1, a2, a3]

    def start(k, slot):
        rs = row0 + k * _ROWS
        cps = []
        for d in range(_D):
            cps.append(pltpu.async_copy(
                pred_hbm.at[b, d, pl.ds(rs, _ROWS), :],
                fbufs[slot][d], sems[slot]))
        cps.append(pltpu.async_copy(
            bin_hbm.at[b, pl.ds(rs, _ROWS), :], fbufs[slot][_D], sems[slot]))
        cps.append(pltpu.async_copy(
            inst_hbm.at[b, pl.ds(rs, _ROWS), :], ibufs[slot], sems[slot]))
        return cps

    zero16 = jnp.zeros((16,), jnp.float32)
    for fb in faccs:
        for i in range(_FACC // 16):
            fb[pl.ds(i * 16, 16)] = zero16
    lane_base = jnp.arange(16, dtype=jnp.int32) * _STRIDE

    vacc = (zero16,) * (2 * _NLAB)

    cps = start(0, 0)
    for k in range(_NCH):
        nxt = start(k + 1, (k + 1) % 2) if k + 1 < _NCH else []
        for cp in cps:
            cp.wait()
        slot = k % 2

        @plsc.parallel_loop(0, _CH // 16, unroll=8, carry=vacc)
        def gbody(g, cv, slot=slot):
            r = g >> 5
            o16 = (g & 31) * 16
            iv = ibufs[slot][r, pl.ds(o16, 16)]
            bv = fbufs[slot][_D][r, pl.ds(o16, 16)]
            xs = [fbufs[slot][d][r, pl.ds(o16, 16)] * bv for d in range(_D)]
            q = xs[0] * xs[0] + xs[1] * xs[1] + xs[2] * xs[2] + xs[3] * xs[3]
            idx = lane_base + iv
            for d in range(_D):
                plsc.addupdate_scatter(faccs[d], [idx], xs[d])
            new = list(cv)
            for cl in range(_NLAB):
                m = iv == cl
                new[cl] = new[cl] + jnp.where(m, q, 0.0)
                new[_NLAB + cl] = new[_NLAB + cl] + jnp.where(m, 1.0, 0.0)
            return tuple(new)

        vacc = gbody
        cps = nxt

    for cl in range(_NLAB):
        for d in range(_D):
            v = plsc.load_gather(faccs[d], [lane_base + cl])
            res[pl.ds((cl * _NF + d) * 16, 16)] = v
        res[pl.ds((cl * _NF + _D) * 16, 16)] = vacc[cl]
        res[pl.ds((cl * _NF + _D + 1) * 16, 16)] = vacc[_NLAB + cl]
    pltpu.sync_copy(res, out_hbm.at[pl.ds(wid * _RES, _RES)])


def _stage1(pred, binary_label, inst_i32):
    mesh = plsc.VectorSubcoreMesh(core_axis_name="c", subcore_axis_name="s")
    fslot = ([pltpu.VMEM((_ROWS, 512), jnp.float32)] * (_D + 1)
             + [pltpu.VMEM((_ROWS, 512), jnp.int32)])
    return pl.kernel(
        _sc_body,
        out_type=jax.ShapeDtypeStruct((_W * _RES,), jnp.float32),
        mesh=mesh,
        compiler_params=pltpu.CompilerParams(
            needs_layout_passes=False, use_tc_tiling_on_sc=True),
        scratch_types=fslot + fslot
        + [pltpu.VMEM((_FACC,), jnp.float32)] * _D
        + [
            pltpu.VMEM((_RES,), jnp.float32),
            pltpu.SemaphoreType.DMA,
            pltpu.SemaphoreType.DMA,
        ],
    )(pred, binary_label, inst_i32)


def _epilogue_body(p_ref, o_ref):
    P = p_ref[...]
    Ps = jnp.sum(P, axis=2)
    rows = lax.broadcasted_iota(jnp.int32, (_B, _W), 0)
    cols = lax.broadcasted_iota(jnp.int32, (_B, _W), 1)
    M = (cols // _WPB == rows).astype(jnp.float32)
    stats = jnp.dot(M, Ps, preferred_element_type=jnp.float32)

    def col(k):
        return stats[:, k:k + 1]

    cnt = [col(cl * _NF + 5) for cl in range(_NLAB)]
    qv = [col(cl * _NF + 4) for cl in range(_NLAB)]
    sv = [[col(cl * _NF + d) for d in range(_D)] for cl in range(_NLAB)]

    present = [cnt[cl] > 0.0 for cl in range(_NLAB)]
    pf = [jnp.where(present[cl], 1.0, 0.0) for cl in range(_NLAB)]
    var_sum = jnp.zeros((_B, 1), jnp.float32)
    mu = []
    for cl in range(_NLAB):
        cs = jnp.maximum(cnt[cl], 1.0)
        m = [sv[cl][d] / cs for d in range(_D)]
        mu.append(m)
        mdots = sum(m[d] * sv[cl][d] for d in range(_D))
        msq = sum(m[d] * m[d] for d in range(_D))
        sumsq = qv[cl] - 2.0 * mdots + cnt[cl] * msq
        nrm = jnp.sqrt(jnp.maximum(sumsq, 0.0))
        delta = jnp.where(nrm > _DELTA_V, nrm - _DELTA_V, 0.0)
        var_sum = var_sum + jnp.where(present[cl], delta * delta, 0.0)

    Cf = jnp.zeros((_B, 1), jnp.float32)
    for cl in range(_NLAB):
        Cf = jnp.maximum(Cf, jnp.where(present[cl], float(cl), 0.0))
    L_var = var_sum / Cf

    running = jnp.zeros((_B, 1), jnp.float32)
    rank = []
    for cl in range(_NLAB):
        running = running + pf[cl]
        rank.append(running - 1.0)
    npres = running
    mr = []
    for r in range(_NLAB):
        md = [jnp.zeros((_B, 1), jnp.float32) for _ in range(_D)]
        for cl in range(_NLAB):
            selw = jnp.where((rank[cl] == float(r)) & present[cl], 1.0, 0.0)
            for d in range(_D):
                md[d] = md[d] + mu[cl][d] * selw
        mr.append(md)

    dist_sum = jnp.zeros((_B, 1), jnp.float32)
    for a in range(_NLAB):
        for b2 in range(a + 1, _NLAB):
            dsq = sum(jnp.square(mr[a][d] - mr[b2][d]) for d in range(_D))
            dd = jnp.sqrt(dsq)
            term = jnp.square(jnp.maximum(_DELTA_D - dd, 0.0))
            valid = (Cf > float(a)) & (Cf > float(b2)) & (npres > 1.5)
            dist_sum = dist_sum + 2.0 * jnp.where(valid, term, 0.0)

    total = jnp.sum(L_var + dist_sum) / float(_B)
    o_ref[...] = jnp.reshape(total, (1, 1))


def _stage2(partials):
    return pl.pallas_call(
        _epilogue_body,
        out_shape=jax.ShapeDtypeStruct((1, 1), jnp.float32),
    )(partials)


@jax.jit
def kernel(pred, binary_label, instance_label):
    partials = _stage1(pred, binary_label, instance_label.astype(jnp.int32))
    out = _stage2(partials.reshape(_W, _NST, 16))
    return out[0, 0]


# speedup vs baseline: 2.5028x; 2.1115x over previous
import functools

import jax
import jax.numpy as jnp
from jax import lax
from jax.experimental import pallas as pl
from jax.experimental.pallas import tpu as pltpu
from jax.experimental.pallas import tpu_sc as plsc

_DELTA_V = 0.5
_DELTA_D = 3.0
_NLAB = 5

_B = 8
_D = 4
_N = 512 * 512
_W = 32
_WPB = _W // _B
_PPW = _N // _WPB
_ROWS = 16
_CH = _ROWS * 512
_NCH = _PPW // _CH
_NF = 6
_NST = _NLAB * _NF
_STRIDE = 7
_FACC = 16 * _STRIDE
_RES = _NST * 16


def _sc_body(pred_hbm, bin_hbm, inst_hbm, out_hbm,
             p0a, p1a, p2a, p3a, bna, ina,
             p0b, p1b, p2b, p3b, bnb, inb,
             a0, a1, a2, a3, res, sem0, sem1):
    cid = lax.axis_index("c")
    sid = lax.axis_index("s")
    wid = sid * 2 + cid
    b = wid // _WPB
    sl = wid % _WPB
    row0 = sl * (512 // _WPB)
    sems = [sem0, sem1]
    fbufs = [[p0a, p1a, p2a, p3a, bna], [p0b, p1b, p2b, p3b, bnb]]
    ibufs = [ina, inb]
    faccs = [a0, a1, a2, a3]

    def start(k, slot):
        rs = row0 + k * _ROWS
        cps = []
        for d in range(_D):
            cps.append(pltpu.async_copy(
                pred_hbm.at[b, d, pl.ds(rs, _ROWS), :],
                fbufs[slot][d], sems[slot]))
        cps.append(pltpu.async_copy(
            bin_hbm.at[b, pl.ds(rs, _ROWS), :], fbufs[slot][_D], sems[slot]))
        cps.append(pltpu.async_copy(
            inst_hbm.at[b, pl.ds(rs, _ROWS), :], ibufs[slot], sems[slot]))
        return cps

    zero16 = jnp.zeros((16,), jnp.float32)
    for fb in faccs:
        for i in range(_FACC // 16):
            fb[pl.ds(i * 16, 16)] = zero16
    lane_base = jnp.arange(16, dtype=jnp.int32) * _STRIDE
    ones16 = jnp.ones((16,), jnp.float32)

    vacc = (zero16,) * (2 * _NLAB)

    cps = start(0, 0)
    for k in range(_NCH):
        nxt = start(k + 1, (k + 1) % 2) if k + 1 < _NCH else []
        for cp in cps:
            cp.wait()
        slot = k % 2

        @plsc.parallel_loop(0, _CH // 16, unroll=8, carry=vacc)
        def gbody(g, cv, slot=slot):
            r = g >> 5
            o16 = (g & 31) * 16
            iv = ibufs[slot][r, pl.ds(o16, 16)]
            bv = fbufs[slot][_D][r, pl.ds(o16, 16)]
            xs = [fbufs[slot][d][r, pl.ds(o16, 16)] * bv for d in range(_D)]
            q = xs[0] * xs[0] + xs[1] * xs[1] + xs[2] * xs[2] + xs[3] * xs[3]
            idx = lane_base + iv
            plsc.addupdate_scatter(faccs[0], [idx], xs[0])
            plsc.addupdate_scatter(faccs[1], [idx], xs[1])
            plsc.addupdate_scatter(faccs[2], [idx], q)
            plsc.addupdate_scatter(faccs[3], [idx], ones16)
            new = list(cv)
            for cl in range(_NLAB):
                m = iv == cl
                new[cl] = new[cl] + jnp.where(m, xs[2], 0.0)
                new[_NLAB + cl] = new[_NLAB + cl] + jnp.where(m, xs[3], 0.0)
            return tuple(new)

        vacc = gbody
        cps = nxt

    scat_field = {0: 0, 1: 1, 2: _D, 3: _D + 1}
    for cl in range(_NLAB):
        for j in range(4):
            v = plsc.load_gather(faccs[j], [lane_base + cl])
            res[pl.ds((cl * _NF + scat_field[j]) * 16, 16)] = v
        res[pl.ds((cl * _NF + 2) * 16, 16)] = vacc[cl]
        res[pl.ds((cl * _NF + 3) * 16, 16)] = vacc[_NLAB + cl]
    pltpu.sync_copy(res, out_hbm.at[pl.ds(wid * _RES, _RES)])


def _stage1(pred, binary_label, inst_i32):
    mesh = plsc.VectorSubcoreMesh(core_axis_name="c", subcore_axis_name="s")
    fslot = ([pltpu.VMEM((_ROWS, 512), jnp.float32)] * (_D + 1)
             + [pltpu.VMEM((_ROWS, 512), jnp.int32)])
    return pl.kernel(
        _sc_body,
        out_type=jax.ShapeDtypeStruct((_W * _RES,), jnp.float32),
        mesh=mesh,
        compiler_params=pltpu.CompilerParams(
            needs_layout_passes=False, use_tc_tiling_on_sc=True),
        scratch_types=fslot + fslot
        + [pltpu.VMEM((_FACC,), jnp.float32)] * _D
        + [
            pltpu.VMEM((_RES,), jnp.float32),
            pltpu.SemaphoreType.DMA,
            pltpu.SemaphoreType.DMA,
        ],
    )(pred, binary_label, inst_i32)


def _epilogue_body(p_ref, o_ref):
    P = p_ref[...]
    Ps = jnp.sum(P, axis=2)
    rows = lax.broadcasted_iota(jnp.int32, (_B, _W), 0)
    cols = lax.broadcasted_iota(jnp.int32, (_B, _W), 1)
    M = (cols // _WPB == rows).astype(jnp.float32)
    stats = jnp.dot(M, Ps, preferred_element_type=jnp.float32)

    def col(k):
        return stats[:, k:k + 1]

    cnt = [col(cl * _NF + 5) for cl in range(_NLAB)]
    qv = [col(cl * _NF + 4) for cl in range(_NLAB)]
    sv = [[col(cl * _NF + d) for d in range(_D)] for cl in range(_NLAB)]

    present = [cnt[cl] > 0.0 for cl in range(_NLAB)]
    pf = [jnp.where(present[cl], 1.0, 0.0) for cl in range(_NLAB)]
    var_sum = jnp.zeros((_B, 1), jnp.float32)
    mu = []
    for cl in range(_NLAB):
        cs = jnp.maximum(cnt[cl], 1.0)
        m = [sv[cl][d] / cs for d in range(_D)]
        mu.append(m)
        mdots = sum(m[d] * sv[cl][d] for d in range(_D))
        msq = sum(m[d] * m[d] for d in range(_D))
        sumsq = qv[cl] - 2.0 * mdots + cnt[cl] * msq
        nrm = jnp.sqrt(jnp.maximum(sumsq, 0.0))
        delta = jnp.where(nrm > _DELTA_V, nrm - _DELTA_V, 0.0)
        var_sum = var_sum + jnp.where(present[cl], delta * delta, 0.0)

    Cf = jnp.zeros((_B, 1), jnp.float32)
    for cl in range(_NLAB):
        Cf = jnp.maximum(Cf, jnp.where(present[cl], float(cl), 0.0))
    L_var = var_sum / Cf

    running = jnp.zeros((_B, 1), jnp.float32)
    rank = []
    for cl in range(_NLAB):
        running = running + pf[cl]
        rank.append(running - 1.0)
    npres = running
    mr = []
    for r in range(_NLAB):
        md = [jnp.zeros((_B, 1), jnp.float32) for _ in range(_D)]
        for cl in range(_NLAB):
            selw = jnp.where((rank[cl] == float(r)) & present[cl], 1.0, 0.0)
            for d in range(_D):
                md[d] = md[d] + mu[cl][d] * selw
        mr.append(md)

    dist_sum = jnp.zeros((_B, 1), jnp.float32)
    for a in range(_NLAB):
        for b2 in range(a + 1, _NLAB):
            dsq = sum(jnp.square(mr[a][d] - mr[b2][d]) for d in range(_D))
            dd = jnp.sqrt(dsq)
            term = jnp.square(jnp.maximum(_DELTA_D - dd, 0.0))
            valid = (Cf > float(a)) & (Cf > float(b2)) & (npres > 1.5)
            dist_sum = dist_sum + 2.0 * jnp.where(valid, term, 0.0)

    total = jnp.sum(L_var + dist_sum) / float(_B)
    o_ref[...] = jnp.reshape(total, (1, 1))


def _stage2(partials):
    return pl.pallas_call(
        _epilogue_body,
        out_shape=jax.ShapeDtypeStruct((1, 1), jnp.float32),
    )(partials)


@jax.jit
def kernel(pred, binary_label, instance_label):
    partials = _stage1(pred, binary_label, instance_label.astype(jnp.int32))
    out = _stage2(partials.reshape(_W, _NST, 16))
    return out[0, 0]


# speedup vs baseline: 2.5344x; 1.0126x over previous
import functools

import jax
import jax.numpy as jnp
from jax import lax
from jax.experimental import pallas as pl
from jax.experimental.pallas import tpu as pltpu
from jax.experimental.pallas import tpu_sc as plsc

_DELTA_V = 0.5
_DELTA_D = 3.0
_NLAB = 5

_B = 8
_D = 4
_N = 512 * 512
_W = 32
_WPB = _W // _B
_PPW = _N // _WPB
_ROWS = 16
_CH = _ROWS * 512
_NCH = _PPW // _CH
_NF = 6
_NST = _NLAB * _NF
_STRIDE = 7
_FACC = 16 * _STRIDE
_RES = _NST * 16


def _sc_body(pred_hbm, bin_hbm, inst_hbm, out_hbm,
             p0a, p1a, p2a, p3a, bna, ina,
             p0b, p1b, p2b, p3b, bnb, inb,
             a0, a1, a2, a3, res, sem0, sem1):
    cid = lax.axis_index("c")
    sid = lax.axis_index("s")
    wid = sid * 2 + cid
    b = wid // _WPB
    sl = wid % _WPB
    row0 = sl * (512 // _WPB)
    sems = [sem0, sem1]
    fbufs = [[p0a, p1a, p2a, p3a, bna], [p0b, p1b, p2b, p3b, bnb]]
    ibufs = [ina, inb]
    faccs = [a0, a1, a2, a3]

    def start(k, slot):
        rs = row0 + k * _ROWS
        cps = []
        for d in range(_D):
            cps.append(pltpu.async_copy(
                pred_hbm.at[b, d, pl.ds(rs, _ROWS), :],
                fbufs[slot][d], sems[slot]))
        cps.append(pltpu.async_copy(
            bin_hbm.at[b, pl.ds(rs, _ROWS), :], fbufs[slot][_D], sems[slot]))
        cps.append(pltpu.async_copy(
            inst_hbm.at[b, pl.ds(rs, _ROWS), :], ibufs[slot], sems[slot]))
        return cps

    zero16 = jnp.zeros((16,), jnp.float32)
    for fb in faccs:
        for i in range(_FACC // 16):
            fb[pl.ds(i * 16, 16)] = zero16
    lane_base = jnp.arange(16, dtype=jnp.int32) * _STRIDE
    ones16 = jnp.ones((16,), jnp.float32)

    vacc = (zero16,) * (2 * _NLAB)

    cps = start(0, 0)
    for k in range(_NCH):
        nxt = start(k + 1, (k + 1) % 2) if k + 1 < _NCH else []
        for cp in cps:
            cp.wait()
        slot = k % 2

        @plsc.parallel_loop(0, _CH // 16, unroll=16, carry=vacc)
        def gbody(g, cv, slot=slot):
            r = g >> 5
            o16 = (g & 31) * 16
            iv = ibufs[slot][r, pl.ds(o16, 16)]
            bv = fbufs[slot][_D][r, pl.ds(o16, 16)]
            xs = [fbufs[slot][d][r, pl.ds(o16, 16)] * bv for d in range(_D)]
            q = xs[0] * xs[0] + xs[1] * xs[1] + xs[2] * xs[2] + xs[3] * xs[3]
            idx = lane_base + iv
            plsc.addupdate_scatter(faccs[0], [idx], xs[0])
            plsc.addupdate_scatter(faccs[1], [idx], xs[1])
            plsc.addupdate_scatter(faccs[2], [idx], q)
            plsc.addupdate_scatter(faccs[3], [idx], ones16)
            new = list(cv)
            for cl in range(_NLAB):
                m = iv == cl
                new[cl] = new[cl] + jnp.where(m, xs[2], 0.0)
                new[_NLAB + cl] = new[_NLAB + cl] + jnp.where(m, xs[3], 0.0)
            return tuple(new)

        vacc = gbody
        cps = nxt

    scat_field = {0: 0, 1: 1, 2: _D, 3: _D + 1}
    for cl in range(_NLAB):
        for j in range(4):
            v = plsc.load_gather(faccs[j], [lane_base + cl])
            res[pl.ds((cl * _NF + scat_field[j]) * 16, 16)] = v
        res[pl.ds((cl * _NF + 2) * 16, 16)] = vacc[cl]
        res[pl.ds((cl * _NF + 3) * 16, 16)] = vacc[_NLAB + cl]
    pltpu.sync_copy(res, out_hbm.at[pl.ds(wid * _RES, _RES)])


def _stage1(pred, binary_label, inst_i32):
    mesh = plsc.VectorSubcoreMesh(core_axis_name="c", subcore_axis_name="s")
    fslot = ([pltpu.VMEM((_ROWS, 512), jnp.float32)] * (_D + 1)
             + [pltpu.VMEM((_ROWS, 512), jnp.int32)])
    return pl.kernel(
        _sc_body,
        out_type=jax.ShapeDtypeStruct((_W * _RES,), jnp.float32),
        mesh=mesh,
        compiler_params=pltpu.CompilerParams(
            needs_layout_passes=False, use_tc_tiling_on_sc=True),
        scratch_types=fslot + fslot
        + [pltpu.VMEM((_FACC,), jnp.float32)] * _D
        + [
            pltpu.VMEM((_RES,), jnp.float32),
            pltpu.SemaphoreType.DMA,
            pltpu.SemaphoreType.DMA,
        ],
    )(pred, binary_label, inst_i32)


def _epilogue_body(p_ref, o_ref):
    P = p_ref[...]
    Ps = jnp.sum(P, axis=2)
    rows = lax.broadcasted_iota(jnp.int32, (_B, _W), 0)
    cols = lax.broadcasted_iota(jnp.int32, (_B, _W), 1)
    M = (cols // _WPB == rows).astype(jnp.float32)
    stats = jnp.dot(M, Ps, preferred_element_type=jnp.float32)

    def col(k):
        return stats[:, k:k + 1]

    cnt = [col(cl * _NF + 5) for cl in range(_NLAB)]
    qv = [col(cl * _NF + 4) for cl in range(_NLAB)]
    sv = [[col(cl * _NF + d) for d in range(_D)] for cl in range(_NLAB)]

    present = [cnt[cl] > 0.0 for cl in range(_NLAB)]
    pf = [jnp.where(present[cl], 1.0, 0.0) for cl in range(_NLAB)]
    var_sum = jnp.zeros((_B, 1), jnp.float32)
    mu = []
    for cl in range(_NLAB):
        cs = jnp.maximum(cnt[cl], 1.0)
        m = [sv[cl][d] / cs for d in range(_D)]
        mu.append(m)
        mdots = sum(m[d] * sv[cl][d] for d in range(_D))
        msq = sum(m[d] * m[d] for d in range(_D))
        sumsq = qv[cl] - 2.0 * mdots + cnt[cl] * msq
        nrm = jnp.sqrt(jnp.maximum(sumsq, 0.0))
        delta = jnp.where(nrm > _DELTA_V, nrm - _DELTA_V, 0.0)
        var_sum = var_sum + jnp.where(present[cl], delta * delta, 0.0)

    Cf = jnp.zeros((_B, 1), jnp.float32)
    for cl in range(_NLAB):
        Cf = jnp.maximum(Cf, jnp.where(present[cl], float(cl), 0.0))
    L_var = var_sum / Cf

    running = jnp.zeros((_B, 1), jnp.float32)
    rank = []
    for cl in range(_NLAB):
        running = running + pf[cl]
        rank.append(running - 1.0)
    npres = running
    mr = []
    for r in range(_NLAB):
        md = [jnp.zeros((_B, 1), jnp.float32) for _ in range(_D)]
        for cl in range(_NLAB):
            selw = jnp.where((rank[cl] == float(r)) & present[cl], 1.0, 0.0)
            for d in range(_D):
                md[d] = md[d] + mu[cl][d] * selw
        mr.append(md)

    dist_sum = jnp.zeros((_B, 1), jnp.float32)
    for a in range(_NLAB):
        for b2 in range(a + 1, _NLAB):
            dsq = sum(jnp.square(mr[a][d] - mr[b2][d]) for d in range(_D))
            dd = jnp.sqrt(dsq)
            term = jnp.square(jnp.maximum(_DELTA_D - dd, 0.0))
            valid = (Cf > float(a)) & (Cf > float(b2)) & (npres > 1.5)
            dist_sum = dist_sum + 2.0 * jnp.where(valid, term, 0.0)

    total = jnp.sum(L_var + dist_sum) / float(_B)
    o_ref[...] = jnp.reshape(total, (1, 1))


def _stage2(partials):
    return pl.pallas_call(
        _epilogue_body,
        out_shape=jax.ShapeDtypeStruct((1, 1), jnp.float32),
    )(partials)


@jax.jit
def kernel(pred, binary_label, instance_label):
    partials = _stage1(pred, binary_label, instance_label.astype(jnp.int32))
    out = _stage2(partials.reshape(_W, _NST, 16))
    return out[0, 0]
